# baseline (speedup 1.0000x reference)
import functools

import jax
import jax.numpy as jnp
from jax import lax
from jax.experimental import pallas as pl
from jax.experimental.pallas import tpu as pltpu
from jax.experimental.pallas import tpu_sc as plsc

N = 10000
NP = 10240
D = 256
H = 8
C = 32
HC = H * C
T = 3
E = 160000
EP = 163840
NTILES = 32
EPT = EP // NTILES
EB = 128
NB = EPT // EB
EPT2 = EP // 16
NB2 = EPT2 // EB
RPT = NP // 16
NP2 = 10112
RPT2 = NP2 // 16

BN1 = 400
BN3 = 632


def _tc1_body(x_ref, nt_ref, wt_ref, bt_ref, wg_ref, asrc_ref, adst_ref,
              u_ref, v_ref, h0_ref, h1_ref):
    xb = x_ref[...]
    nt = nt_ref[...]
    ps = []
    for t in range(T):
        p = lax.dot_general(xb, wt_ref[t], (((1,), (1,)), ((), ())),
                            preferred_element_type=jnp.float32)
        ps.append(p + bt_ref[pl.ds(t, 1), :])
    xp = jnp.where(nt == 0, ps[0], jnp.where(nt == 1, ps[1], ps[2]))
    h = lax.dot_general(xp, wg_ref[...], (((1,), (1,)), ((), ())),
                        preferred_element_type=jnp.float32)
    h0_ref[...] = h[:, :128]
    h1_ref[...] = h[:, 128:]
    u_ref[...] = lax.dot_general(h, asrc_ref[...], (((1,), (0,)), ((), ())),
                                 preferred_element_type=jnp.float32)
    v_ref[...] = lax.dot_general(h, adst_ref[...], (((1,), (0,)), ((), ())),
                                 preferred_element_type=jnp.float32)


_tc1 = pl.pallas_call(
    _tc1_body,
    grid=(N // BN1,),
    in_specs=[
        pl.BlockSpec((BN1, D), lambda i: (i, 0)),
        pl.BlockSpec((BN1, 1), lambda i: (i, 0)),
        pl.BlockSpec((T, D, D), lambda i: (0, 0, 0)),
        pl.BlockSpec((8, D), lambda i: (0, 0)),
        pl.BlockSpec((HC, D), lambda i: (0, 0)),
        pl.BlockSpec((HC, 16), lambda i: (0, 0)),
        pl.BlockSpec((HC, 16), lambda i: (0, 0)),
    ],
    out_specs=[
        pl.BlockSpec((BN1, 16), lambda i: (i, 0)),
        pl.BlockSpec((BN1, 16), lambda i: (i, 0)),
        pl.BlockSpec((BN1, 128), lambda i: (i, 0)),
        pl.BlockSpec((BN1, 128), lambda i: (i, 0)),
    ],
    out_shape=[
        jax.ShapeDtypeStruct((N, 16), jnp.float32),
        jax.ShapeDtypeStruct((N, 16), jnp.float32),
        jax.ShapeDtypeStruct((N, 128), jnp.float32),
        jax.ShapeDtypeStruct((N, 128), jnp.float32),
    ],
)


def _sca_body(src_h, dst_h, u_h, v_h, ex_h, s0_h, s1_h,
              si2, di3, uv2, vv2, ex1, ex2, zs_v, s_sh,
              sxi, sg, sst, ssc):
    c = lax.axis_index("c")
    s = lax.axis_index("s")
    wid = s * 2 + c
    base = wid * EPT
    EBW = EB * 16

    def zrow(i, _):
        zs_v[i] = jnp.zeros((16,), jnp.float32)
        return 0
    lax.fori_loop(0, RPT, zrow, 0)
    pltpu.sync_copy(zs_v, s_sh.at[pl.ds(s * RPT, RPT)])
    plsc.subcore_barrier()

    def r2(j):
        return lax.rem(j, 2)

    def r3(j):
        return lax.rem(j, 3)

    def issue_idx(j):
        k2, k3, eb = r2(j), r3(j), base + j * EB
        pltpu.async_copy(src_h.at[pl.ds(eb, EB)], si2.at[k2], sxi.at[k2])
        pltpu.async_copy(dst_h.at[pl.ds(eb, EB)], di3.at[k3], sxi.at[k2])

    def wait_idx(j):
        k2, k3, eb = r2(j), r3(j), base + j * EB
        pltpu.make_async_copy(src_h.at[pl.ds(eb, EB)], si2.at[k2],
                              sxi.at[k2]).wait()
        pltpu.make_async_copy(dst_h.at[pl.ds(eb, EB)], di3.at[k3],
                              sxi.at[k2]).wait()

    def issue_gat(j):
        k2, k3 = r2(j), r3(j)
        pltpu.async_copy(u_h.at[si2.at[k2]], uv2.at[k2], sg.at[k2])
        pltpu.async_copy(v_h.at[di3.at[k3]], vv2.at[k2], sg.at[k2])

    def wait_gat(j):
        k2, k3 = r2(j), r3(j)
        pltpu.make_async_copy(u_h.at[si2.at[k2]], uv2.at[k2],
                              sg.at[k2]).wait()
        pltpu.make_async_copy(v_h.at[di3.at[k3]], vv2.at[k2],
                              sg.at[k2]).wait()

    def compute(j):
        k2, eb = r2(j), base + j * EB
        uvb = uv2.at[k2]
        vvb = vv2.at[k2]
        e1b = ex1.at[k2]
        e2b = ex2.at[k2]

        def cbody(i, _):
            e = uvb[i] + vvb[i]
            e = jnp.where(e < 0.0, e * jnp.float32(0.2), e)
            ex = jnp.exp(e)
            scale = jnp.where(eb + i < E, jnp.float32(1.0), jnp.float32(0.0))
            exm = ex * lax.broadcast(scale, (16,))
            e1b[pl.ds(i * 16, 16)] = exm
            e2b[i] = exm
            return 0
        lax.fori_loop(0, EB, cbody, 0, unroll=2)

    def issue_out(j):
        k2, k3, eb = r2(j), r3(j), base + j * EB
        pltpu.async_copy(ex1.at[k2], ex_h.at[pl.ds(eb * 16, EBW)], sst.at[k2])
        pltpu.async_copy(ex2.at[k2], s_sh.at[di3.at[k3]], ssc.at[k2],
                         add=True)

    def wait_sst(j):
        k2, eb = r2(j), base + j * EB
        pltpu.make_async_copy(ex1.at[k2], ex_h.at[pl.ds(eb * 16, EBW)],
                              sst.at[k2]).wait()

    def wait_ssc(j):
        k2, k3 = r2(j), r3(j)
        pltpu.make_async_copy(ex2.at[k2], s_sh.at[di3.at[k3]],
                              ssc.at[k2]).wait()

    pltpu.sync_copy(src_h.at[pl.ds(base, EB)], si2.at[0])
    pltpu.sync_copy(dst_h.at[pl.ds(base, EB)], di3.at[0])
    issue_gat(0)
    issue_idx(1)

    wait_gat(0)
    wait_idx(1)
    issue_gat(1)
    compute(0)
    issue_out(0)
    issue_idx(2)

    wait_gat(1)
    wait_idx(2)
    issue_gat(2)
    compute(1)
    issue_out(1)
    wait_ssc(0)
    issue_idx(3)

    def steady(j, _):
        wait_gat(j)
        wait_idx(j + 1)
        issue_gat(j + 1)
        wait_sst(j - 2)
        compute(j)
        issue_out(j)
        wait_ssc(j - 1)
        issue_idx(j + 2)
        return 0
    lax.fori_loop(2, NB - 2, steady, 0)

    j = NB - 2
    wait_gat(j)
    wait_idx(j + 1)
    issue_gat(j + 1)
    wait_sst(j - 2)
    compute(j)
    issue_out(j)
    wait_ssc(j - 1)

    j = NB - 1
    wait_gat(j)
    wait_sst(j - 2)
    compute(j)
    issue_out(j)
    wait_ssc(j - 1)

    wait_sst(NB - 2)
    wait_sst(NB - 1)
    wait_ssc(NB - 1)
    plsc.subcore_barrier()

    r0 = s * RPT

    @pl.when(c == 0)
    def _():
        pltpu.sync_copy(s_sh.at[pl.ds(r0, RPT)], s0_h.at[pl.ds(r0, RPT)])

    @pl.when(c == 1)
    def _():
        pltpu.sync_copy(s_sh.at[pl.ds(r0, RPT)], s1_h.at[pl.ds(r0, RPT)])


_sca = functools.partial(
    pl.kernel,
    mesh=plsc.VectorSubcoreMesh(core_axis_name="c", subcore_axis_name="s"),
    compiler_params=pltpu.CompilerParams(use_tc_tiling_on_sc=False),
    out_type=[
        jax.ShapeDtypeStruct((EP * 16,), jnp.float32),
        jax.ShapeDtypeStruct((NP, 16), jnp.float32),
        jax.ShapeDtypeStruct((NP, 16), jnp.float32),
    ],
    scratch_types=[
        pltpu.VMEM((2, EB), jnp.int32),
        pltpu.VMEM((3, EB), jnp.int32),
        pltpu.VMEM((2, EB, 16), jnp.float32),
        pltpu.VMEM((2, EB, 16), jnp.float32),
        pltpu.VMEM((2, EB * 16), jnp.float32),
        pltpu.VMEM((2, EB, 16), jnp.float32),
        pltpu.VMEM((RPT, 16), jnp.float32),
        pltpu.VMEM_SHARED((NP, 16), jnp.float32),
        pltpu.SemaphoreType.DMA((2,)),
        pltpu.SemaphoreType.DMA((2,)),
        pltpu.SemaphoreType.DMA((2,)),
        pltpu.SemaphoreType.DMA((2,)),
    ],
)(_sca_body)


def _scb_body(src_h, dst_h, ex_h, h01_h, og_h,
              si2, di3, ex2, h2, zb_v, o_sh, sx, shm, ssm):
    c = lax.axis_index("c")
    s = lax.axis_index("s")
    base = s * EPT2
    cn = c * N
    goff = c * 4
    EBW = EB * 16

    def zrow(i, _):
        for k in range(8):
            zb_v[i, pl.ds(k * 16, 16)] = jnp.zeros((16,), jnp.float32)
        return 0
    lax.fori_loop(0, 32, zrow, 0)
    for k in range(19):
        pltpu.sync_copy(zb_v, o_sh.at[pl.ds(s * RPT2 + k * 32, 32)])
    pltpu.sync_copy(zb_v.at[pl.ds(0, RPT2 - 19 * 32)],
                    o_sh.at[pl.ds(s * RPT2 + 19 * 32, RPT2 - 19 * 32)])
    plsc.subcore_barrier()

    def r2(j):
        return lax.rem(j, 2)

    def r3(j):
        return lax.rem(j, 3)

    def issue_idx(j):
        k2, k3, eb = r2(j), r3(j), base + j * EB
        pltpu.async_copy(src_h.at[pl.ds(eb, EB)], si2.at[k2], sx.at[k2])
        pltpu.async_copy(dst_h.at[pl.ds(eb, EB)], di3.at[k3], sx.at[k2])
        pltpu.async_copy(ex_h.at[pl.ds(eb * 16, EBW)],
                         ex2.at[k2, pl.ds(0, EBW)], sx.at[k2])

    def wait_idx(j):
        k2, k3, eb = r2(j), r3(j), base + j * EB
        pltpu.make_async_copy(src_h.at[pl.ds(eb, EB)], si2.at[k2],
                              sx.at[k2]).wait()
        pltpu.make_async_copy(dst_h.at[pl.ds(eb, EB)], di3.at[k3],
                              sx.at[k2]).wait()
        pltpu.make_async_copy(ex_h.at[pl.ds(eb * 16, EBW)],
                              ex2.at[k2, pl.ds(0, EBW)], sx.at[k2]).wait()

    def add_cn(j):
        k2 = r2(j)
        for m in range(8):
            sl = pl.ds(m * 16, 16)
            si2[k2, sl] = si2[k2, sl] + lax.broadcast(cn, (16,))

    def issue_hgat(j):
        k2 = r2(j)
        pltpu.async_copy(h01_h.at[si2.at[k2]], h2.at[k2], shm.at[k2])

    def wait_hgat(j):
        k2 = r2(j)
        pltpu.make_async_copy(h01_h.at[si2.at[k2]], h2.at[k2],
                              shm.at[k2]).wait()

    def issue_scat(j):
        k2, k3 = r2(j), r3(j)
        pltpu.async_copy(h2.at[k2], o_sh.at[di3.at[k3]], ssm.at[k2], add=True)

    def wait_scat(j):
        k2, k3 = r2(j), r3(j)
        pltpu.make_async_copy(h2.at[k2], o_sh.at[di3.at[k3]],
                              ssm.at[k2]).wait()

    def compute(j):
        k2 = r2(j)
        hb = h2.at[k2]
        exb = ex2.at[k2]

        def mbody(i, _):
            row = exb[pl.ds(i * 16 + goff, 16)]
            for h4 in range(4):
                ab = lax.broadcast(row[h4], (16,))
                for q in range(2):
                    sl = pl.ds(h4 * 32 + q * 16, 16)
                    hb[i, sl] = hb[i, sl] * ab
            return 0
        lax.fori_loop(0, EB, mbody, 0, unroll=4)

    pltpu.sync_copy(src_h.at[pl.ds(base, EB)], si2.at[0])
    pltpu.sync_copy(dst_h.at[pl.ds(base, EB)], di3.at[0])
    pltpu.sync_copy(ex_h.at[pl.ds(base * 16, EBW)], ex2.at[0, pl.ds(0, EBW)])
    add_cn(0)
    issue_hgat(0)
    issue_idx(1)

    wait_hgat(0)
    wait_idx(1)
    add_cn(1)
    issue_hgat(1)
    compute(0)
    issue_scat(0)
    issue_idx(2)

    def steady(j, _):
        wait_hgat(j)
        wait_idx(j + 1)
        add_cn(j + 1)
        wait_scat(j - 1)
        issue_hgat(j + 1)
        compute(j)
        issue_scat(j)
        issue_idx(j + 2)
        return 0
    lax.fori_loop(1, NB2 - 2, steady, 0)

    j = NB2 - 2
    wait_hgat(j)
    wait_idx(j + 1)
    add_cn(j + 1)
    wait_scat(j - 1)
    issue_hgat(j + 1)
    compute(j)
    issue_scat(j)

    j = NB2 - 1
    wait_hgat(j)
    wait_scat(j - 1)
    compute(j)
    issue_scat(j)
    wait_scat(NB2 - 1)
    plsc.subcore_barrier()

    r0 = s * RPT2
    pltpu.sync_copy(o_sh.at[pl.ds(r0, RPT2)],
                    og_h.at[pl.ds(c * NP2 + r0, RPT2)])


_scb = functools.partial(
    pl.kernel,
    mesh=plsc.VectorSubcoreMesh(core_axis_name="c", subcore_axis_name="s"),
    compiler_params=pltpu.CompilerParams(use_tc_tiling_on_sc=False),
    out_type=[
        jax.ShapeDtypeStruct((2 * NP2, 128), jnp.float32),
    ],
    scratch_types=[
        pltpu.VMEM((2, EB), jnp.int32),
        pltpu.VMEM((3, EB), jnp.int32),
        pltpu.VMEM((2, EB * 16 + 16), jnp.float32),
        pltpu.VMEM((2, EB, 128), jnp.float32),
        pltpu.VMEM((32, 128), jnp.float32),
        pltpu.VMEM_SHARED((NP2, 128), jnp.float32),
        pltpu.SemaphoreType.DMA((2,)),
        pltpu.SemaphoreType.DMA((2,)),
        pltpu.SemaphoreType.DMA((2,)),
    ],
)(_scb_body)


def _tc3_body(a_ref, b_ref, s0_ref, s1_ref, r_ref, bias_ref, o_ref):
    num = jnp.concatenate([a_ref[...], b_ref[...]], axis=1)
    sden = s0_ref[...] + s1_ref[...]
    den = lax.dot_general(sden, r_ref[...], (((1,), (0,)), ((), ())),
                          preferred_element_type=jnp.float32) + 1e-16
    o_ref[...] = num / den + bias_ref[pl.ds(0, 1), :]


_tc3 = pl.pallas_call(
    _tc3_body,
    grid=(NP2 // BN3,),
    in_specs=[
        pl.BlockSpec((BN3, 128), lambda i: (i, 0)),
        pl.BlockSpec((BN3, 128), lambda i: (NP2 // BN3 + i, 0)),
        pl.BlockSpec((BN3, 16), lambda i: (i, 0)),
        pl.BlockSpec((BN3, 16), lambda i: (i, 0)),
        pl.BlockSpec((16, HC), lambda i: (0, 0)),
        pl.BlockSpec((8, HC), lambda i: (0, 0)),
    ],
    out_specs=pl.BlockSpec((BN3, HC), lambda i: (i, 0)),
    out_shape=jax.ShapeDtypeStruct((NP2, HC), jnp.float32),
)


def kernel(x, Wt, bt, Wg, att_src, att_dst, bias, edge_index, node_types):
    f32 = jnp.float32
    eyeH = jnp.eye(H, dtype=f32)
    a_src_m = (eyeH[:, None, :] * att_src[:, :, None]).reshape(HC, H)
    a_dst_m = (eyeH[:, None, :] * att_dst[:, :, None]).reshape(HC, H)
    acat_src = jnp.concatenate([a_src_m, a_src_m], axis=1)
    acat_dst = jnp.concatenate([a_dst_m, a_dst_m], axis=1)
    bt8 = jnp.zeros((8, D), f32).at[:T].set(bt)
    nt2 = node_types.reshape(N, 1)
    src = jnp.zeros((EP,), jnp.int32).at[:E].set(edge_index[0])
    dst = jnp.zeros((EP,), jnp.int32).at[:E].set(edge_index[1])

    u, v, h0, h1 = _tc1(x, nt2, Wt, bt8, Wg, acat_src, acat_dst)
    ex, s0, s1 = _sca(src, dst, u, v)
    h01 = jnp.concatenate([h0, h1], axis=0)
    og, = _scb(src, dst, ex, h01)

    rexp = jnp.repeat(jnp.eye(16, dtype=f32)[:, :H], C, axis=1)
    bias2 = jnp.broadcast_to(bias[None, :], (8, HC))
    outp = _tc3(og, og, s0, s1, rexp, bias2)
    return outp[:N]


# speedup vs baseline: 1.0492x; 1.0492x over previous
import functools

import jax
import jax.numpy as jnp
from jax import lax
from jax.experimental import pallas as pl
from jax.experimental.pallas import tpu as pltpu
from jax.experimental.pallas import tpu_sc as plsc

N = 10000
NP = 10240
D = 256
H = 8
C = 32
HC = H * C
T = 3
E = 160000
EP = 163840
NTILES = 32
EPT = EP // NTILES
EB = 128
NB = EPT // EB
EPT2 = EP // 16
NB2 = EPT2 // EB
RPT = NP // 16
NP2 = 10112
RPT2 = NP2 // 16

BN1 = 400
BN3 = 632


def _tc1_body(x_ref, nt_ref, wt_ref, bt_ref, wg_ref, asrc_ref, adst_ref,
              u_ref, v_ref, h0_ref, h1_ref):
    xb = x_ref[...]
    nt = nt_ref[...]
    ps = []
    for t in range(T):
        p = lax.dot_general(xb, wt_ref[t], (((1,), (1,)), ((), ())),
                            preferred_element_type=jnp.float32)
        ps.append(p + bt_ref[pl.ds(t, 1), :])
    xp = jnp.where(nt == 0, ps[0], jnp.where(nt == 1, ps[1], ps[2]))
    h = lax.dot_general(xp, wg_ref[...], (((1,), (1,)), ((), ())),
                        preferred_element_type=jnp.float32)
    h0_ref[...] = h[:, :128].astype(jnp.bfloat16)
    h1_ref[...] = h[:, 128:].astype(jnp.bfloat16)
    u_ref[...] = lax.dot_general(h, asrc_ref[...], (((1,), (0,)), ((), ())),
                                 preferred_element_type=jnp.float32)
    v_ref[...] = lax.dot_general(h, adst_ref[...], (((1,), (0,)), ((), ())),
                                 preferred_element_type=jnp.float32)


_tc1 = pl.pallas_call(
    _tc1_body,
    grid=(N // BN1,),
    in_specs=[
        pl.BlockSpec((BN1, D), lambda i: (i, 0)),
        pl.BlockSpec((BN1, 1), lambda i: (i, 0)),
        pl.BlockSpec((T, D, D), lambda i: (0, 0, 0)),
        pl.BlockSpec((8, D), lambda i: (0, 0)),
        pl.BlockSpec((HC, D), lambda i: (0, 0)),
        pl.BlockSpec((HC, 16), lambda i: (0, 0)),
        pl.BlockSpec((HC, 16), lambda i: (0, 0)),
    ],
    out_specs=[
        pl.BlockSpec((BN1, 16), lambda i: (i, 0)),
        pl.BlockSpec((BN1, 16), lambda i: (i, 0)),
        pl.BlockSpec((BN1, 128), lambda i: (i, 0)),
        pl.BlockSpec((BN1, 128), lambda i: (i, 0)),
    ],
    out_shape=[
        jax.ShapeDtypeStruct((N, 16), jnp.float32),
        jax.ShapeDtypeStruct((N, 16), jnp.float32),
        jax.ShapeDtypeStruct((N, 128), jnp.bfloat16),
        jax.ShapeDtypeStruct((N, 128), jnp.bfloat16),
    ],
)


def _sca_body(src_h, dst_h, u_h, v_h, ex_h, s0_h, s1_h,
              si2, di3, uv2, vv2, ex1, ex2, zs_v, s_sh,
              sxi, sg, sst, ssc):
    c = lax.axis_index("c")
    s = lax.axis_index("s")
    wid = s * 2 + c
    base = wid * EPT
    EBW = EB * 16

    def zrow(i, _):
        zs_v[i] = jnp.zeros((16,), jnp.float32)
        return 0
    lax.fori_loop(0, RPT, zrow, 0)
    pltpu.sync_copy(zs_v, s_sh.at[pl.ds(s * RPT, RPT)])
    plsc.subcore_barrier()

    def r2(j):
        return lax.rem(j, 2)

    def r3(j):
        return lax.rem(j, 3)

    def issue_idx(j):
        k2, k3, eb = r2(j), r3(j), base + j * EB
        pltpu.async_copy(src_h.at[pl.ds(eb, EB)], si2.at[k2], sxi.at[k2])
        pltpu.async_copy(dst_h.at[pl.ds(eb, EB)], di3.at[k3], sxi.at[k2])

    def wait_idx(j):
        k2, k3, eb = r2(j), r3(j), base + j * EB
        pltpu.make_async_copy(src_h.at[pl.ds(eb, EB)], si2.at[k2],
                              sxi.at[k2]).wait()
        pltpu.make_async_copy(dst_h.at[pl.ds(eb, EB)], di3.at[k3],
                              sxi.at[k2]).wait()

    def issue_gat(j):
        k2, k3 = r2(j), r3(j)
        pltpu.async_copy(u_h.at[si2.at[k2]], uv2.at[k2], sg.at[k2])
        pltpu.async_copy(v_h.at[di3.at[k3]], vv2.at[k2], sg.at[k2])

    def wait_gat(j):
        k2, k3 = r2(j), r3(j)
        pltpu.make_async_copy(u_h.at[si2.at[k2]], uv2.at[k2],
                              sg.at[k2]).wait()
        pltpu.make_async_copy(v_h.at[di3.at[k3]], vv2.at[k2],
                              sg.at[k2]).wait()

    def compute(j):
        k2, eb = r2(j), base + j * EB
        uvb = uv2.at[k2]
        vvb = vv2.at[k2]
        e1b = ex1.at[k2]
        e2b = ex2.at[k2]

        def cbody(i, _):
            e = uvb[i] + vvb[i]
            e = jnp.where(e < 0.0, e * jnp.float32(0.2), e)
            ex = jnp.exp(e)
            scale = jnp.where(eb + i < E, jnp.float32(1.0), jnp.float32(0.0))
            exm = ex * lax.broadcast(scale, (16,))
            e1b[pl.ds(i * 16, 16)] = exm
            e2b[i] = exm
            return 0
        lax.fori_loop(0, EB, cbody, 0, unroll=2)

    def issue_out(j):
        k2, k3, eb = r2(j), r3(j), base + j * EB
        pltpu.async_copy(ex1.at[k2], ex_h.at[pl.ds(eb * 16, EBW)], sst.at[k2])
        pltpu.async_copy(ex2.at[k2], s_sh.at[di3.at[k3]], ssc.at[k2],
                         add=True)

    def wait_sst(j):
        k2, eb = r2(j), base + j * EB
        pltpu.make_async_copy(ex1.at[k2], ex_h.at[pl.ds(eb * 16, EBW)],
                              sst.at[k2]).wait()

    def wait_ssc(j):
        k2, k3 = r2(j), r3(j)
        pltpu.make_async_copy(ex2.at[k2], s_sh.at[di3.at[k3]],
                              ssc.at[k2]).wait()

    pltpu.sync_copy(src_h.at[pl.ds(base, EB)], si2.at[0])
    pltpu.sync_copy(dst_h.at[pl.ds(base, EB)], di3.at[0])
    issue_gat(0)
    issue_idx(1)

    wait_gat(0)
    wait_idx(1)
    issue_gat(1)
    compute(0)
    issue_out(0)
    issue_idx(2)

    wait_gat(1)
    wait_idx(2)
    issue_gat(2)
    compute(1)
    issue_out(1)
    wait_ssc(0)
    issue_idx(3)

    def steady(j, _):
        wait_gat(j)
        wait_idx(j + 1)
        issue_gat(j + 1)
        wait_sst(j - 2)
        compute(j)
        issue_out(j)
        wait_ssc(j - 1)
        issue_idx(j + 2)
        return 0
    lax.fori_loop(2, NB - 2, steady, 0)

    j = NB - 2
    wait_gat(j)
    wait_idx(j + 1)
    issue_gat(j + 1)
    wait_sst(j - 2)
    compute(j)
    issue_out(j)
    wait_ssc(j - 1)

    j = NB - 1
    wait_gat(j)
    wait_sst(j - 2)
    compute(j)
    issue_out(j)
    wait_ssc(j - 1)

    wait_sst(NB - 2)
    wait_sst(NB - 1)
    wait_ssc(NB - 1)
    plsc.subcore_barrier()

    r0 = s * RPT

    @pl.when(c == 0)
    def _():
        pltpu.sync_copy(s_sh.at[pl.ds(r0, RPT)], s0_h.at[pl.ds(r0, RPT)])

    @pl.when(c == 1)
    def _():
        pltpu.sync_copy(s_sh.at[pl.ds(r0, RPT)], s1_h.at[pl.ds(r0, RPT)])


_sca = functools.partial(
    pl.kernel,
    mesh=plsc.VectorSubcoreMesh(core_axis_name="c", subcore_axis_name="s"),
    compiler_params=pltpu.CompilerParams(use_tc_tiling_on_sc=False),
    out_type=[
        jax.ShapeDtypeStruct((EP * 16,), jnp.float32),
        jax.ShapeDtypeStruct((NP, 16), jnp.float32),
        jax.ShapeDtypeStruct((NP, 16), jnp.float32),
    ],
    scratch_types=[
        pltpu.VMEM((2, EB), jnp.int32),
        pltpu.VMEM((3, EB), jnp.int32),
        pltpu.VMEM((2, EB, 16), jnp.float32),
        pltpu.VMEM((2, EB, 16), jnp.float32),
        pltpu.VMEM((2, EB * 16), jnp.float32),
        pltpu.VMEM((2, EB, 16), jnp.float32),
        pltpu.VMEM((RPT, 16), jnp.float32),
        pltpu.VMEM_SHARED((NP, 16), jnp.float32),
        pltpu.SemaphoreType.DMA((2,)),
        pltpu.SemaphoreType.DMA((2,)),
        pltpu.SemaphoreType.DMA((2,)),
        pltpu.SemaphoreType.DMA((2,)),
    ],
)(_sca_body)


def _scb_body(src_h, dst_h, ex_h, h01_h, og_h,
              si2, di3, ex2, h2, msg_v, zb_v, o_sh, sx, shm, ssm):
    c = lax.axis_index("c")
    s = lax.axis_index("s")
    base = s * EPT2
    cn = c * N
    goff = c * 4
    EBW = EB * 16

    def zrow(i, _):
        for k in range(8):
            zb_v[i, pl.ds(k * 16, 16)] = jnp.zeros((16,), jnp.float32)
        return 0
    lax.fori_loop(0, 32, zrow, 0)
    for k in range(19):
        pltpu.sync_copy(zb_v, o_sh.at[pl.ds(s * RPT2 + k * 32, 32)])
    pltpu.sync_copy(zb_v.at[pl.ds(0, RPT2 - 19 * 32)],
                    o_sh.at[pl.ds(s * RPT2 + 19 * 32, RPT2 - 19 * 32)])
    plsc.subcore_barrier()

    def r2(j):
        return lax.rem(j, 2)

    def r3(j):
        return lax.rem(j, 3)

    def issue_idx(j):
        k2, k3, eb = r2(j), r3(j), base + j * EB
        pltpu.async_copy(src_h.at[pl.ds(eb, EB)], si2.at[k2], sx.at[k2])
        pltpu.async_copy(dst_h.at[pl.ds(eb, EB)], di3.at[k3], sx.at[k2])
        pltpu.async_copy(ex_h.at[pl.ds(eb * 16, EBW)],
                         ex2.at[k2, pl.ds(0, EBW)], sx.at[k2])

    def wait_idx(j):
        k2, k3, eb = r2(j), r3(j), base + j * EB
        pltpu.make_async_copy(src_h.at[pl.ds(eb, EB)], si2.at[k2],
                              sx.at[k2]).wait()
        pltpu.make_async_copy(dst_h.at[pl.ds(eb, EB)], di3.at[k3],
                              sx.at[k2]).wait()
        pltpu.make_async_copy(ex_h.at[pl.ds(eb * 16, EBW)],
                              ex2.at[k2, pl.ds(0, EBW)], sx.at[k2]).wait()

    def add_cn(j):
        k2 = r2(j)
        for m in range(8):
            sl = pl.ds(m * 16, 16)
            si2[k2, sl] = si2[k2, sl] + lax.broadcast(cn, (16,))

    def issue_hgat(j):
        k2 = r2(j)
        pltpu.async_copy(h01_h.at[si2.at[k2]], h2.at[k2], shm.at[k2])

    def wait_hgat(j):
        k2 = r2(j)
        pltpu.make_async_copy(h01_h.at[si2.at[k2]], h2.at[k2],
                              shm.at[k2]).wait()

    def issue_scat(j):
        k2, k3 = r2(j), r3(j)
        pltpu.async_copy(msg_v, o_sh.at[di3.at[k3]], ssm.at[k2], add=True)

    def wait_scat(j):
        k2, k3 = r2(j), r3(j)
        pltpu.make_async_copy(msg_v, o_sh.at[di3.at[k3]],
                              ssm.at[k2]).wait()

    def compute(j):
        k2 = r2(j)
        hb = h2.at[k2]
        exb = ex2.at[k2]

        def mbody(i, _):
            row = exb[pl.ds(i * 16 + goff, 16)]
            for h4 in range(4):
                ab = lax.broadcast(row[h4], (16,))
                seg = hb[i, pl.ds(h4 * 32, 32)]
                pa, pb = plsc.unpack(seg, format=plsc.PackFormat.INTERLEAVED)
                msg_v[i, pl.ds(h4 * 32, 16)] = pa * ab
                msg_v[i, pl.ds(h4 * 32 + 16, 16)] = pb * ab
            return 0
        lax.fori_loop(0, EB, mbody, 0, unroll=4)

    pltpu.sync_copy(src_h.at[pl.ds(base, EB)], si2.at[0])
    pltpu.sync_copy(dst_h.at[pl.ds(base, EB)], di3.at[0])
    pltpu.sync_copy(ex_h.at[pl.ds(base * 16, EBW)], ex2.at[0, pl.ds(0, EBW)])
    add_cn(0)
    issue_hgat(0)
    issue_idx(1)

    wait_hgat(0)
    wait_idx(1)
    add_cn(1)
    issue_hgat(1)
    compute(0)
    issue_scat(0)
    issue_idx(2)

    def steady(j, _):
        wait_hgat(j)
        wait_idx(j + 1)
        add_cn(j + 1)
        wait_scat(j - 1)
        issue_hgat(j + 1)
        compute(j)
        issue_scat(j)
        issue_idx(j + 2)
        return 0
    lax.fori_loop(1, NB2 - 2, steady, 0)

    j = NB2 - 2
    wait_hgat(j)
    wait_idx(j + 1)
    add_cn(j + 1)
    wait_scat(j - 1)
    issue_hgat(j + 1)
    compute(j)
    issue_scat(j)

    j = NB2 - 1
    wait_hgat(j)
    wait_scat(j - 1)
    compute(j)
    issue_scat(j)
    wait_scat(NB2 - 1)
    plsc.subcore_barrier()

    r0 = s * RPT2
    pltpu.sync_copy(o_sh.at[pl.ds(r0, RPT2)],
                    og_h.at[pl.ds(c * NP2 + r0, RPT2)])


_scb = functools.partial(
    pl.kernel,
    mesh=plsc.VectorSubcoreMesh(core_axis_name="c", subcore_axis_name="s"),
    compiler_params=pltpu.CompilerParams(use_tc_tiling_on_sc=False, needs_layout_passes=False),
    out_type=[
        jax.ShapeDtypeStruct((2 * NP2, 128), jnp.float32),
    ],
    scratch_types=[
        pltpu.VMEM((2, EB), jnp.int32),
        pltpu.VMEM((3, EB), jnp.int32),
        pltpu.VMEM((2, EB * 16 + 16), jnp.float32),
        pltpu.VMEM((2, EB, 128), jnp.bfloat16),
        pltpu.VMEM((EB, 128), jnp.float32),
        pltpu.VMEM((32, 128), jnp.float32),
        pltpu.VMEM_SHARED((NP2, 128), jnp.float32),
        pltpu.SemaphoreType.DMA((2,)),
        pltpu.SemaphoreType.DMA((2,)),
        pltpu.SemaphoreType.DMA((2,)),
    ],
)(_scb_body)


def _tc3_body(a_ref, b_ref, s0_ref, s1_ref, r_ref, bias_ref, o_ref):
    num = jnp.concatenate([a_ref[...], b_ref[...]], axis=1)
    sden = s0_ref[...] + s1_ref[...]
    den = lax.dot_general(sden, r_ref[...], (((1,), (0,)), ((), ())),
                          preferred_element_type=jnp.float32) + 1e-16
    o_ref[...] = num / den + bias_ref[pl.ds(0, 1), :]


_tc3 = pl.pallas_call(
    _tc3_body,
    grid=(NP2 // BN3,),
    in_specs=[
        pl.BlockSpec((BN3, 128), lambda i: (i, 0)),
        pl.BlockSpec((BN3, 128), lambda i: (NP2 // BN3 + i, 0)),
        pl.BlockSpec((BN3, 16), lambda i: (i, 0)),
        pl.BlockSpec((BN3, 16), lambda i: (i, 0)),
        pl.BlockSpec((16, HC), lambda i: (0, 0)),
        pl.BlockSpec((8, HC), lambda i: (0, 0)),
    ],
    out_specs=pl.BlockSpec((BN3, HC), lambda i: (i, 0)),
    out_shape=jax.ShapeDtypeStruct((NP2, HC), jnp.float32),
)


def kernel(x, Wt, bt, Wg, att_src, att_dst, bias, edge_index, node_types):
    f32 = jnp.float32
    eyeH = jnp.eye(H, dtype=f32)
    a_src_m = (eyeH[:, None, :] * att_src[:, :, None]).reshape(HC, H)
    a_dst_m = (eyeH[:, None, :] * att_dst[:, :, None]).reshape(HC, H)
    blk = jnp.arange(C // 2)
    inner = jnp.stack([blk, blk + C // 2], axis=1).reshape(-1)
    perm = (jnp.arange(H)[:, None] * C + inner[None, :]).reshape(-1)
    Wg = Wg[perm]
    a_src_m = a_src_m[perm]
    a_dst_m = a_dst_m[perm]
    acat_src = jnp.concatenate([a_src_m, a_src_m], axis=1)
    acat_dst = jnp.concatenate([a_dst_m, a_dst_m], axis=1)
    bt8 = jnp.zeros((8, D), f32).at[:T].set(bt)
    nt2 = node_types.reshape(N, 1)
    src = jnp.zeros((EP,), jnp.int32).at[:E].set(edge_index[0])
    dst = jnp.zeros((EP,), jnp.int32).at[:E].set(edge_index[1])

    u, v, h0, h1 = _tc1(x, nt2, Wt, bt8, Wg, acat_src, acat_dst)
    ex, s0, s1 = _sca(src, dst, u, v)
    h01 = jnp.concatenate([h0, h1], axis=0)
    og, = _scb(src, dst, ex, h01)

    rexp = jnp.repeat(jnp.eye(16, dtype=f32)[:, :H], C, axis=1)
    bias2 = jnp.broadcast_to(bias[None, :], (8, HC))
    outp = _tc3(og, og, s0, s1, rexp, bias2)
    return outp[:N]
